# baseline (speedup 1.0000x reference)
import jax
import jax.numpy as jnp
from jax import lax
from jax.experimental import pallas as pl
from jax.experimental.pallas import tpu as pltpu
from jax.experimental.pallas import tpu_sc as plsc

N = 10000
D = 128
E = 320000

NC = 2
NS = 16
B = 128
E2 = E + N
EPAD = -(-E2 // (NS * B * 4)) * (NS * B * 4)
NP = 10240
RPT = NP // NS
NBLK1 = EPAD // (NS * B)
NBLK2 = EPAD // (NC * NS * B)
CH = 512
NCHUNK = NBLK1 * B // CH

_MESH = plsc.VectorSubcoreMesh(core_axis_name="c", subcore_axis_name="s",
                               num_cores=NC, num_subcores=NS)


def _zero_rows(rows):
    z16 = jnp.zeros((16,), jnp.float32)

    def zr(i, _):
        rows[i // (D // 16), pl.ds((i % (D // 16)) * 16, 16)] = z16
        return 0

    lax.fori_loop(0, B * D // 16, zr, 0)


def _init_acc(rows, acc, sid):
    for j in range(RPT // B):
        pltpu.sync_copy(rows, acc.at[pl.ds(sid * RPT + j * B, B)])


def _edge_loop(val_hbm, src_hbm, gbase, dst_hbm, dbase,
               S, SC_, D2, SD, RW, SR, SS, acc, nblk, cnt_ctx=None):
    def req_s(b, p):
        pltpu.async_copy(src_hbm.at[pl.ds(gbase + b * B, B)], S[p], SC_[p])

    def req_d(b, p):
        pltpu.async_copy(dst_hbm.at[pl.ds(dbase + b * B, B)], D2[p], SD[p])

    def wait_scat(p, r):
        pltpu.make_async_copy(RW[r], acc.at[D2[p]], SS[r]).wait()

    def gather(b, p, r, guarded=True):
        pltpu.make_async_copy(src_hbm.at[pl.ds(gbase + b * B, B)],
                              S[p], SC_[p]).wait()
        pdrain = (p + 2) % 4
        if guarded:
            @pl.when(b >= 2)
            def _():
                wait_scat(pdrain, r)
        elif b >= 2:
            wait_scat(pdrain, r)
        pltpu.async_copy(val_hbm.at[S[p]], RW[r], SR[r])

    o16 = jnp.ones((16,), jnp.float32)

    def scat(b, p, r):
        pltpu.make_async_copy(val_hbm.at[S[p]], RW[r], SR[r]).wait()
        pltpu.make_async_copy(dst_hbm.at[pl.ds(dbase + b * B, B)],
                              D2[p], SD[p]).wait()
        pltpu.async_copy(RW[r], acc.at[D2[p]], SS[r])
        if cnt_ctx is not None:
            for e in range(B // 16):
                plsc.addupdate_scatter(
                    cnt_ctx, [D2[p][pl.ds(e * 16, 16)]], o16)

    for b in range(4):
        req_s(b, b)
    for b in range(2):
        req_d(b, b)
    gather(0, 0, 0, guarded=False)

    main = nblk // 4 - 1

    def quad(j, _):
        for k in range(4):
            b = 4 * j + k
            scat(b, k, k % 2)
            gather(b + 1, (k + 1) % 4, (k + 1) % 2)
            req_s(b + 4, k)
            req_d(b + 2, (k + 2) % 4)
        return 0

    lax.fori_loop(0, main, quad, 0)

    for bb in range(4 * main, nblk):
        k = bb % 4
        scat(bb, k, k % 2)
        if bb + 1 < nblk:
            gather(bb + 1, (k + 1) % 4, (k + 1) % 2, guarded=False)
        if bb + 4 < nblk:
            req_s(bb + 4, k)
        if bb + 2 < nblk:
            req_d(bb + 2, (k + 2) % 4)

    wait_scat((nblk - 2) % 4, (nblk - 2) % 2)
    wait_scat((nblk - 1) % 4, (nblk - 1) % 2)


def _seg1_body(x_hbm, h_hbm, src_hbm, dst_hbm, sx_hbm, sh_hbm, cnt_hbm,
               s0, s1, s2, s3, d0, d1, d2, d3, rows0, rows1,
               cnt_v, acc,
               sc0, sc1, sc2, sc3, sd0, sd1, sd2, sd3,
               sr0, sr1, ss0, ss1):
    cid = lax.axis_index("c")
    sid = lax.axis_index("s")
    z16 = jnp.zeros((16,), jnp.float32)

    _zero_rows(rows0)
    _init_acc(rows0, acc, sid)

    @pl.when(cid == 0)
    def _():
        def zc(i, _):
            cnt_v[pl.ds(i * 16, 16)] = z16
            return 0

        lax.fori_loop(0, NP // 16, zc, 0)

    plsc.subcore_barrier()

    tb = sid * (NBLK1 * B)
    bufs = ([s0, s1, s2, s3], [sc0, sc1, sc2, sc3],
            [d0, d1, d2, d3], [sd0, sd1, sd2, sd3],
            [rows0, rows1], [sr0, sr1], [ss0, ss1])

    @pl.when(cid == 0)
    def _():
        _edge_loop(x_hbm, src_hbm, tb, dst_hbm, tb, *bufs,
                   acc, NBLK1, cnt_ctx=cnt_v)
        pltpu.sync_copy(cnt_v, cnt_hbm.at[sid])

    @pl.when(cid == 1)
    def _():
        _edge_loop(h_hbm, src_hbm, tb, dst_hbm, tb, *bufs,
                   acc, NBLK1)

    plsc.subcore_barrier()

    @pl.when(cid == 0)
    def _():
        pltpu.sync_copy(acc.at[pl.ds(sid * RPT, RPT)],
                        sx_hbm.at[pl.ds(sid * RPT, RPT)])

    @pl.when(cid == 1)
    def _():
        pltpu.sync_copy(acc.at[pl.ds(sid * RPT, RPT)],
                        sh_hbm.at[pl.ds(sid * RPT, RPT)])


_seg1 = pl.kernel(
    _seg1_body,
    out_type=[jax.ShapeDtypeStruct((NP, D), jnp.float32),
              jax.ShapeDtypeStruct((NP, D), jnp.float32),
              jax.ShapeDtypeStruct((NS, NP), jnp.float32)],
    mesh=_MESH,
    scratch_types=[
        pltpu.VMEM((B,), jnp.int32),
        pltpu.VMEM((B,), jnp.int32),
        pltpu.VMEM((B,), jnp.int32),
        pltpu.VMEM((B,), jnp.int32),
        pltpu.VMEM((B,), jnp.int32),
        pltpu.VMEM((B,), jnp.int32),
        pltpu.VMEM((B,), jnp.int32),
        pltpu.VMEM((B,), jnp.int32),
        pltpu.VMEM((B, D), jnp.float32),
        pltpu.VMEM((B, D), jnp.float32),
        pltpu.VMEM((NP,), jnp.float32),
        pltpu.VMEM_SHARED((NP, D), jnp.float32),
    ] + [pltpu.SemaphoreType.DMA] * 12,
    compiler_params=pltpu.CompilerParams(needs_layout_passes=False),
)


def _seg2_body(rh_hbm, src_hbm, dst_hbm, sa_hbm, sb_hbm,
               s0, s1, s2, s3, d0, d1, d2, d3, rows0, rows1, acc,
               sc0, sc1, sc2, sc3, sd0, sd1, sd2, sd3,
               sr0, sr1, ss0, ss1):
    cid = lax.axis_index("c")
    sid = lax.axis_index("s")

    _zero_rows(rows0)
    _init_acc(rows0, acc, sid)

    plsc.subcore_barrier()

    wbase = (cid * NS + sid) * (NBLK2 * B)
    _edge_loop(rh_hbm, src_hbm, wbase, dst_hbm, wbase,
               [s0, s1, s2, s3], [sc0, sc1, sc2, sc3],
               [d0, d1, d2, d3], [sd0, sd1, sd2, sd3],
               [rows0, rows1], [sr0, sr1], [ss0, ss1],
               acc, NBLK2)

    plsc.subcore_barrier()

    @pl.when(cid == 0)
    def _():
        pltpu.sync_copy(acc.at[pl.ds(sid * RPT, RPT)],
                        sa_hbm.at[pl.ds(sid * RPT, RPT)])

    @pl.when(cid == 1)
    def _():
        pltpu.sync_copy(acc.at[pl.ds(sid * RPT, RPT)],
                        sb_hbm.at[pl.ds(sid * RPT, RPT)])


_seg2 = pl.kernel(
    _seg2_body,
    out_type=[jax.ShapeDtypeStruct((NP, D), jnp.float32),
              jax.ShapeDtypeStruct((NP, D), jnp.float32)],
    mesh=_MESH,
    scratch_types=[
        pltpu.VMEM((B,), jnp.int32),
        pltpu.VMEM((B,), jnp.int32),
        pltpu.VMEM((B,), jnp.int32),
        pltpu.VMEM((B,), jnp.int32),
        pltpu.VMEM((B,), jnp.int32),
        pltpu.VMEM((B,), jnp.int32),
        pltpu.VMEM((B,), jnp.int32),
        pltpu.VMEM((B,), jnp.int32),
        pltpu.VMEM((B, D), jnp.float32),
        pltpu.VMEM((B, D), jnp.float32),
        pltpu.VMEM_SHARED((NP, D), jnp.float32),
    ] + [pltpu.SemaphoreType.DMA] * 12,
    compiler_params=pltpu.CompilerParams(needs_layout_passes=False),
)

BR = 1024


def _tc1_body(sx, sh, cnt, hp, wx, wh, z_o, rh_o, pxh_o):
    cs = jnp.sum(jnp.transpose(cnt[...]), axis=1, keepdims=True)
    inv = 1.0 / jnp.maximum(cs, 1.0)
    mx = sx[...] * inv
    mh = sh[...] * inv
    px = jax.nn.relu(jnp.dot(mx, wx[...], preferred_element_type=jnp.float32))
    ph = jax.nn.relu(jnp.dot(mh, wh[...], preferred_element_type=jnp.float32))
    z = jax.nn.sigmoid(px[:, :D] + ph[:, :D])
    r = jax.nn.sigmoid(px[:, D:2 * D] + ph[:, D:2 * D])
    z_o[...] = z
    rh_o[...] = r * hp[...]
    pxh_o[...] = px[:, 2 * D:]


def _tc1(sx, sh, cnt16, hp, wx, wh):
    blk = lambda i: (i, 0)
    w0 = lambda i: (0, 0)
    return pl.pallas_call(
        _tc1_body,
        grid=(NP // BR,),
        in_specs=[
            pl.BlockSpec((BR, D), blk),
            pl.BlockSpec((BR, D), blk),
            pl.BlockSpec((NS, BR), lambda i: (0, i)),
            pl.BlockSpec((BR, D), blk),
            pl.BlockSpec((D, 3 * D), w0),
            pl.BlockSpec((D, 2 * D), w0),
        ],
        out_specs=[pl.BlockSpec((BR, D), blk)] * 3,
        out_shape=[jax.ShapeDtypeStruct((N, D), jnp.float32)] * 3,
    )(sx, sh, cnt16, hp, wx, wh)


def _tc2_body(sa, sb, cnt, hp, z, pxh, whh, out):
    cs = jnp.sum(jnp.transpose(cnt[...]), axis=1, keepdims=True)
    inv = 1.0 / jnp.maximum(cs, 1.0)
    m = (sa[...] + sb[...]) * inv
    ph = jax.nn.relu(jnp.dot(m, whh[...], preferred_element_type=jnp.float32))
    ht = jnp.tanh(pxh[...] + ph)
    zz = z[...]
    out[...] = zz * hp[...] + (1.0 - zz) * ht


def _tc2(sa, sb, cnt16, hp, z, pxh, whh):
    blk = lambda i: (i, 0)
    w0 = lambda i: (0, 0)
    return pl.pallas_call(
        _tc2_body,
        grid=(NP // BR,),
        in_specs=[
            pl.BlockSpec((BR, D), blk),
            pl.BlockSpec((BR, D), blk),
            pl.BlockSpec((NS, BR), lambda i: (0, i)),
            pl.BlockSpec((BR, D), blk),
            pl.BlockSpec((BR, D), blk),
            pl.BlockSpec((BR, D), blk),
            pl.BlockSpec((D, D), w0),
        ],
        out_specs=pl.BlockSpec((BR, D), blk),
        out_shape=jax.ShapeDtypeStruct((N, D), jnp.float32),
    )(sa, sb, cnt16, hp, z, pxh, whh)


def kernel(x, edge_index, h_prev, W_xz, W_hz, W_xr, W_hr, W_xh, W_hh):
    loops = jnp.arange(N, dtype=jnp.int32)
    pad = EPAD - E2
    pk = jnp.arange(pad, dtype=jnp.int32)
    row = jnp.concatenate([edge_index[0], loops, pk % N])
    col = jnp.concatenate([edge_index[1], loops, N + pk % (NP - N)])

    sx, sh, cnt16 = _seg1(x, h_prev, row, col)
    wx = jnp.concatenate([W_xz, W_xr, W_xh], axis=1)
    wh = jnp.concatenate([W_hz, W_hr], axis=1)

    z, rh, pxh = _tc1(sx, sh, cnt16, h_prev, wx, wh)

    s2a, s2b = _seg2(rh, row, col)

    return _tc2(s2a, s2b, cnt16, h_prev, z, pxh, W_hh)


# speedup vs baseline: 1.1704x; 1.1704x over previous
import jax
import jax.numpy as jnp
from jax import lax
from jax.experimental import pallas as pl
from jax.experimental.pallas import tpu as pltpu
from jax.experimental.pallas import tpu_sc as plsc

N = 10000
D = 128
E = 320000

NC = 2
NS = 16
B = 128
E2 = E + N
EPAD = -(-E2 // (NS * B * 4)) * (NS * B * 4)
NP = 10240
RPT = NP // NS
NBLK1 = EPAD // (NS * B)
NBLK2 = EPAD // (NC * NS * B)
CH = 512
NCHUNK = NBLK1 * B // CH

_MESH = plsc.VectorSubcoreMesh(core_axis_name="c", subcore_axis_name="s",
                               num_cores=NC, num_subcores=NS)


def _zero_rows(rows):
    z16 = jnp.zeros((16,), jnp.float32)

    def zr(i, _):
        rows[i // (D // 16), pl.ds((i % (D // 16)) * 16, 16)] = z16
        return 0

    lax.fori_loop(0, B * D // 16, zr, 0)


def _init_acc(rows, acc, sid):
    for j in range(RPT // B):
        pltpu.sync_copy(rows, acc.at[pl.ds(sid * RPT + j * B, B)])


def _edge_loop(val_hbm, src_hbm, gbase, dst_hbm, dbase,
               S, SC_, D2, SD, RW, SR, acc, nblk, cnt_ctx=None):
    def req_s(b, p):
        pltpu.async_copy(src_hbm.at[pl.ds(gbase + b * B, B)], S[p], SC_[p])

    def req_d(b, p):
        pltpu.async_copy(dst_hbm.at[pl.ds(dbase + b * B, B)], D2[p], SD[p])

    def gather(b, p, r):
        pltpu.make_async_copy(src_hbm.at[pl.ds(gbase + b * B, B)],
                              S[p], SC_[p]).wait()
        pltpu.async_copy(val_hbm.at[S[p]], RW[r], SR[r])

    o16 = jnp.ones((16,), jnp.float32)

    def scat(b, p, r):
        pltpu.make_async_copy(val_hbm.at[S[p]], RW[r], SR[r]).wait()
        pltpu.make_async_copy(dst_hbm.at[pl.ds(dbase + b * B, B)],
                              D2[r], SD[r]).wait()
        pltpu.sync_copy(RW[r], acc.at[D2[r]], add=True)
        if cnt_ctx is not None:
            for e in range(B // 16):
                plsc.addupdate_scatter(
                    cnt_ctx, [D2[r][pl.ds(e * 16, 16)]], o16)

    for b in range(4):
        req_s(b, b)
    for b in range(2):
        req_d(b, b)
    gather(0, 0, 0)

    main = nblk // 4 - 1

    def quad(j, _):
        for k in range(4):
            b = 4 * j + k
            gather(b + 1, (k + 1) % 4, (k + 1) % 2)
            scat(b, k, k % 2)
            req_s(b + 4, k)
            req_d(b + 2, k % 2)
        return 0

    lax.fori_loop(0, main, quad, 0)

    for bb in range(4 * main, nblk):
        b = 4 * main + (bb - 4 * main)
        k = bb % 4
        if bb + 1 < nblk:
            gather(bb + 1, (k + 1) % 4, (k + 1) % 2)
        scat(bb, k, k % 2)
        if bb + 4 < nblk:
            req_s(bb + 4, k)
        if bb + 2 < nblk:
            req_d(bb + 2, k % 2)


def _seg1_body(x_hbm, h_hbm, src_hbm, dst_hbm, sx_hbm, sh_hbm, cnt_hbm,
               s0, s1, s2, s3, d0, d1, rows0, rows1,
               cnt_v, acc,
               sc0, sc1, sc2, sc3, sd0, sd1, sr0, sr1):
    cid = lax.axis_index("c")
    sid = lax.axis_index("s")
    z16 = jnp.zeros((16,), jnp.float32)

    _zero_rows(rows0)
    _init_acc(rows0, acc, sid)

    @pl.when(cid == 0)
    def _():
        def zc(i, _):
            cnt_v[pl.ds(i * 16, 16)] = z16
            return 0

        lax.fori_loop(0, NP // 16, zc, 0)

    plsc.subcore_barrier()

    tb = sid * (NBLK1 * B)
    bufs = ([s0, s1, s2, s3], [sc0, sc1, sc2, sc3],
            [d0, d1], [sd0, sd1], [rows0, rows1], [sr0, sr1])

    @pl.when(cid == 0)
    def _():
        _edge_loop(x_hbm, src_hbm, tb, dst_hbm, tb, *bufs,
                   acc, NBLK1, cnt_ctx=cnt_v)
        pltpu.sync_copy(cnt_v, cnt_hbm.at[sid])

    @pl.when(cid == 1)
    def _():
        _edge_loop(h_hbm, src_hbm, tb, dst_hbm, tb, *bufs,
                   acc, NBLK1)

    plsc.subcore_barrier()

    @pl.when(cid == 0)
    def _():
        pltpu.sync_copy(acc.at[pl.ds(sid * RPT, RPT)],
                        sx_hbm.at[pl.ds(sid * RPT, RPT)])

    @pl.when(cid == 1)
    def _():
        pltpu.sync_copy(acc.at[pl.ds(sid * RPT, RPT)],
                        sh_hbm.at[pl.ds(sid * RPT, RPT)])


_seg1 = pl.kernel(
    _seg1_body,
    out_type=[jax.ShapeDtypeStruct((NP, D), jnp.float32),
              jax.ShapeDtypeStruct((NP, D), jnp.float32),
              jax.ShapeDtypeStruct((NS, NP), jnp.float32)],
    mesh=_MESH,
    scratch_types=[
        pltpu.VMEM((B,), jnp.int32),
        pltpu.VMEM((B,), jnp.int32),
        pltpu.VMEM((B,), jnp.int32),
        pltpu.VMEM((B,), jnp.int32),
        pltpu.VMEM((B,), jnp.int32),
        pltpu.VMEM((B,), jnp.int32),
        pltpu.VMEM((B, D), jnp.float32),
        pltpu.VMEM((B, D), jnp.float32),
        pltpu.VMEM((NP,), jnp.float32),
        pltpu.VMEM_SHARED((NP, D), jnp.float32),
    ] + [pltpu.SemaphoreType.DMA] * 8,
    compiler_params=pltpu.CompilerParams(needs_layout_passes=False),
)


def _seg2_body(rh_hbm, src_hbm, dst_hbm, sa_hbm, sb_hbm,
               s0, s1, s2, s3, d0, d1, rows0, rows1, acc,
               sc0, sc1, sc2, sc3, sd0, sd1, sr0, sr1):
    cid = lax.axis_index("c")
    sid = lax.axis_index("s")

    _zero_rows(rows0)
    _init_acc(rows0, acc, sid)

    plsc.subcore_barrier()

    wbase = (cid * NS + sid) * (NBLK2 * B)
    _edge_loop(rh_hbm, src_hbm, wbase, dst_hbm, wbase,
               [s0, s1, s2, s3], [sc0, sc1, sc2, sc3],
               [d0, d1], [sd0, sd1], [rows0, rows1], [sr0, sr1],
               acc, NBLK2)

    plsc.subcore_barrier()

    @pl.when(cid == 0)
    def _():
        pltpu.sync_copy(acc.at[pl.ds(sid * RPT, RPT)],
                        sa_hbm.at[pl.ds(sid * RPT, RPT)])

    @pl.when(cid == 1)
    def _():
        pltpu.sync_copy(acc.at[pl.ds(sid * RPT, RPT)],
                        sb_hbm.at[pl.ds(sid * RPT, RPT)])


_seg2 = pl.kernel(
    _seg2_body,
    out_type=[jax.ShapeDtypeStruct((NP, D), jnp.float32),
              jax.ShapeDtypeStruct((NP, D), jnp.float32)],
    mesh=_MESH,
    scratch_types=[
        pltpu.VMEM((B,), jnp.int32),
        pltpu.VMEM((B,), jnp.int32),
        pltpu.VMEM((B,), jnp.int32),
        pltpu.VMEM((B,), jnp.int32),
        pltpu.VMEM((B,), jnp.int32),
        pltpu.VMEM((B,), jnp.int32),
        pltpu.VMEM((B, D), jnp.float32),
        pltpu.VMEM((B, D), jnp.float32),
        pltpu.VMEM_SHARED((NP, D), jnp.float32),
    ] + [pltpu.SemaphoreType.DMA] * 8,
    compiler_params=pltpu.CompilerParams(needs_layout_passes=False),
)

BR = 1024


def _tc1_body(sx, sh, cnt, hp, wx, wh, z_o, rh_o, pxh_o, inv_o):
    cs = jnp.sum(jnp.transpose(cnt[...]), axis=1, keepdims=True)
    inv = 1.0 / jnp.maximum(cs, 1.0)
    mx = sx[...] * inv
    mh = sh[...] * inv
    px = jax.nn.relu(jnp.dot(mx, wx[...], preferred_element_type=jnp.float32))
    ph = jax.nn.relu(jnp.dot(mh, wh[...], preferred_element_type=jnp.float32))
    z = jax.nn.sigmoid(px[:, :D] + ph[:, :D])
    r = jax.nn.sigmoid(px[:, D:2 * D] + ph[:, D:2 * D])
    z_o[...] = z
    rh_o[...] = r * hp[...]
    pxh_o[...] = px[:, 2 * D:]
    inv_o[...] = inv


def _tc1(sx, sh, cnt16, hp, wx, wh):
    blk = lambda i: (i, 0)
    w0 = lambda i: (0, 0)
    return pl.pallas_call(
        _tc1_body,
        grid=(NP // BR,),
        in_specs=[
            pl.BlockSpec((BR, D), blk),
            pl.BlockSpec((BR, D), blk),
            pl.BlockSpec((NS, BR), lambda i: (0, i)),
            pl.BlockSpec((BR, D), blk),
            pl.BlockSpec((D, 3 * D), w0),
            pl.BlockSpec((D, 2 * D), w0),
        ],
        out_specs=[pl.BlockSpec((BR, D), blk)] * 3 + [pl.BlockSpec((BR, 1), blk)],
        out_shape=[jax.ShapeDtypeStruct((N, D), jnp.float32)] * 3
        + [jax.ShapeDtypeStruct((N, 1), jnp.float32)],
    )(sx, sh, cnt16, hp, wx, wh)


def _tc2_body(sa, sb, inv, hp, z, pxh, whh, out):
    m = (sa[...] + sb[...]) * inv[...]
    ph = jax.nn.relu(jnp.dot(m, whh[...], preferred_element_type=jnp.float32))
    ht = jnp.tanh(pxh[...] + ph)
    zz = z[...]
    out[...] = zz * hp[...] + (1.0 - zz) * ht


def _tc2(sa, sb, inv, hp, z, pxh, whh):
    blk = lambda i: (i, 0)
    w0 = lambda i: (0, 0)
    return pl.pallas_call(
        _tc2_body,
        grid=(NP // BR,),
        in_specs=[
            pl.BlockSpec((BR, D), blk),
            pl.BlockSpec((BR, D), blk),
            pl.BlockSpec((BR, 1), blk),
            pl.BlockSpec((BR, D), blk),
            pl.BlockSpec((BR, D), blk),
            pl.BlockSpec((BR, D), blk),
            pl.BlockSpec((D, D), w0),
        ],
        out_specs=pl.BlockSpec((BR, D), blk),
        out_shape=jax.ShapeDtypeStruct((N, D), jnp.float32),
    )(sa, sb, inv, hp, z, pxh, whh)


def kernel(x, edge_index, h_prev, W_xz, W_hz, W_xr, W_hr, W_xh, W_hh):
    loops = jnp.arange(N, dtype=jnp.int32)
    pad = EPAD - E2
    pk = jnp.arange(pad, dtype=jnp.int32)
    row = jnp.concatenate([edge_index[0], loops, pk % N])
    col = jnp.concatenate([edge_index[1], loops, N + pk % (NP - N)])

    sx, sh, cnt16 = _seg1(x, h_prev, row, col)
    wx = jnp.concatenate([W_xz, W_xr, W_xh], axis=1)
    wh = jnp.concatenate([W_hz, W_hr], axis=1)

    z, rh, pxh, inv = _tc1(sx, sh, cnt16, h_prev, wx, wh)

    s2a, s2b = _seg2(rh, row, col)

    return _tc2(s2a, s2b, inv, h_prev, z, pxh, W_hh)


# speedup vs baseline: 1.1745x; 1.0035x over previous
import jax
import jax.numpy as jnp
from jax import lax
from jax.experimental import pallas as pl
from jax.experimental.pallas import tpu as pltpu
from jax.experimental.pallas import tpu_sc as plsc

N = 10000
D = 128
E = 320000

NC = 2
NS = 16
B = 128
E2 = E + N
EPAD = -(-E2 // (NS * B * 4)) * (NS * B * 4)
NP = 10240
RPT = NP // NS
NBLK1 = EPAD // (NS * B)
NBLK2 = EPAD // (NC * NS * B)

_MESH = plsc.VectorSubcoreMesh(core_axis_name="c", subcore_axis_name="s",
                               num_cores=NC, num_subcores=NS)


def _zero_rows(rows):
    z16 = jnp.zeros((16,), jnp.float32)

    def zr(i, _):
        rows[i // (D // 16), pl.ds((i % (D // 16)) * 16, 16)] = z16
        return 0

    lax.fori_loop(0, B * D // 16, zr, 0)


def _init_acc(rows, acc, sid):
    for j in range(RPT // B):
        pltpu.sync_copy(rows, acc.at[pl.ds(sid * RPT + j * B, B)])


def _edge_loop(val_hbm, src_hbm, gbase, dst_hbm, dbase,
               S, SC_, D2, SD, RW, SR, acc, nblk, cnt_ctx=None):
    def req_s(b, p):
        pltpu.async_copy(src_hbm.at[pl.ds(gbase + b * B, B)], S[p], SC_[p])

    def req_d(b, p):
        pltpu.async_copy(dst_hbm.at[pl.ds(dbase + b * B, B)], D2[p], SD[p])

    def gather(b, p, r):
        pltpu.make_async_copy(src_hbm.at[pl.ds(gbase + b * B, B)],
                              S[p], SC_[p]).wait()
        pltpu.async_copy(val_hbm.at[S[p]], RW[r], SR[r])

    o16 = jnp.ones((16,), jnp.float32)

    def scat(b, p, r):
        pltpu.make_async_copy(val_hbm.at[S[p]], RW[r], SR[r]).wait()
        pltpu.make_async_copy(dst_hbm.at[pl.ds(dbase + b * B, B)],
                              D2[r], SD[r]).wait()
        pltpu.sync_copy(RW[r], acc.at[D2[r]], add=True)
        if cnt_ctx is not None:
            for e in range(B // 16):
                plsc.addupdate_scatter(
                    cnt_ctx, [D2[r][pl.ds(e * 16, 16)]], o16)

    for b in range(4):
        req_s(b, b)
    for b in range(2):
        req_d(b, b)
    gather(0, 0, 0)

    main = nblk // 4 - 1

    def quad(j, _):
        for k in range(4):
            b = 4 * j + k
            gather(b + 1, (k + 1) % 4, (k + 1) % 2)
            scat(b, k, k % 2)
            req_s(b + 4, k)
            req_d(b + 2, k % 2)
        return 0

    lax.fori_loop(0, main, quad, 0)

    for bb in range(4 * main, nblk):
        k = bb % 4
        if bb + 1 < nblk:
            gather(bb + 1, (k + 1) % 4, (k + 1) % 2)
        scat(bb, k, k % 2)
        if bb + 4 < nblk:
            req_s(bb + 4, k)
        if bb + 2 < nblk:
            req_d(bb + 2, k % 2)


def _seg1_body(x_hbm, h_hbm, src_hbm, dst_hbm, sx_hbm, sh_hbm, cnt_hbm,
               s0, s1, s2, s3, d0, d1, rows0, rows1,
               cnt_v, acc,
               sc0, sc1, sc2, sc3, sd0, sd1, sr0, sr1):
    cid = lax.axis_index("c")
    sid = lax.axis_index("s")
    z16 = jnp.zeros((16,), jnp.float32)

    _zero_rows(rows0)
    _init_acc(rows0, acc, sid)

    @pl.when(cid == 0)
    def _():
        def zc(i, _):
            cnt_v[pl.ds(i * 16, 16)] = z16
            return 0

        lax.fori_loop(0, NP // 16, zc, 0)

    plsc.subcore_barrier()

    tb = sid * (NBLK1 * B)
    bufs = ([s0, s1, s2, s3], [sc0, sc1, sc2, sc3],
            [d0, d1], [sd0, sd1], [rows0, rows1], [sr0, sr1])

    @pl.when(cid == 0)
    def _():
        _edge_loop(x_hbm, src_hbm, tb, dst_hbm, tb, *bufs,
                   acc, NBLK1, cnt_ctx=cnt_v)
        pltpu.sync_copy(cnt_v, cnt_hbm.at[sid])

    @pl.when(cid == 1)
    def _():
        _edge_loop(h_hbm, src_hbm, tb, dst_hbm, tb, *bufs,
                   acc, NBLK1)

    plsc.subcore_barrier()

    @pl.when(cid == 0)
    def _():
        pltpu.sync_copy(acc.at[pl.ds(sid * RPT, RPT)],
                        sx_hbm.at[pl.ds(sid * RPT, RPT)])

    @pl.when(cid == 1)
    def _():
        pltpu.sync_copy(acc.at[pl.ds(sid * RPT, RPT)],
                        sh_hbm.at[pl.ds(sid * RPT, RPT)])


_seg1 = pl.kernel(
    _seg1_body,
    out_type=[jax.ShapeDtypeStruct((NP, D), jnp.float32),
              jax.ShapeDtypeStruct((NP, D), jnp.float32),
              jax.ShapeDtypeStruct((NS, NP), jnp.float32)],
    mesh=_MESH,
    scratch_types=[
        pltpu.VMEM((B,), jnp.int32),
        pltpu.VMEM((B,), jnp.int32),
        pltpu.VMEM((B,), jnp.int32),
        pltpu.VMEM((B,), jnp.int32),
        pltpu.VMEM((B,), jnp.int32),
        pltpu.VMEM((B,), jnp.int32),
        pltpu.VMEM((B, D), jnp.float32),
        pltpu.VMEM((B, D), jnp.float32),
        pltpu.VMEM((NP,), jnp.float32),
        pltpu.VMEM_SHARED((NP, D), jnp.float32),
    ] + [pltpu.SemaphoreType.DMA] * 8,
    compiler_params=pltpu.CompilerParams(needs_layout_passes=False),
)


def _seg2_body(rh_hbm, src_hbm, dst_hbm, sa_hbm, sb_hbm,
               s0, s1, s2, s3, d0, d1, rows0, rows1, acc,
               sc0, sc1, sc2, sc3, sd0, sd1, sr0, sr1):
    cid = lax.axis_index("c")
    sid = lax.axis_index("s")

    _zero_rows(rows0)
    _init_acc(rows0, acc, sid)

    plsc.subcore_barrier()

    wbase = (cid * NS + sid) * (NBLK2 * B)
    _edge_loop(rh_hbm, src_hbm, wbase, dst_hbm, wbase,
               [s0, s1, s2, s3], [sc0, sc1, sc2, sc3],
               [d0, d1], [sd0, sd1], [rows0, rows1], [sr0, sr1],
               acc, NBLK2)

    plsc.subcore_barrier()

    @pl.when(cid == 0)
    def _():
        pltpu.sync_copy(acc.at[pl.ds(sid * RPT, RPT)],
                        sa_hbm.at[pl.ds(sid * RPT, RPT)])

    @pl.when(cid == 1)
    def _():
        pltpu.sync_copy(acc.at[pl.ds(sid * RPT, RPT)],
                        sb_hbm.at[pl.ds(sid * RPT, RPT)])


_seg2 = pl.kernel(
    _seg2_body,
    out_type=[jax.ShapeDtypeStruct((NP, D), jnp.float32),
              jax.ShapeDtypeStruct((NP, D), jnp.float32)],
    mesh=_MESH,
    scratch_types=[
        pltpu.VMEM((B,), jnp.int32),
        pltpu.VMEM((B,), jnp.int32),
        pltpu.VMEM((B,), jnp.int32),
        pltpu.VMEM((B,), jnp.int32),
        pltpu.VMEM((B,), jnp.int32),
        pltpu.VMEM((B,), jnp.int32),
        pltpu.VMEM((B, D), jnp.float32),
        pltpu.VMEM((B, D), jnp.float32),
        pltpu.VMEM_SHARED((NP, D), jnp.float32),
    ] + [pltpu.SemaphoreType.DMA] * 8,
    compiler_params=pltpu.CompilerParams(needs_layout_passes=False),
)

BR = 1024


def _tc1_body(sx, sh, cnt, hp, wx, wh, z_o, rh_o, pxh_o, inv_o):
    cs = jnp.sum(jnp.transpose(cnt[...]), axis=1, keepdims=True)
    inv = 1.0 / jnp.maximum(cs, 1.0)
    mx = sx[...] * inv
    mh = sh[...] * inv
    px = jax.nn.relu(jnp.dot(mx, wx[...], preferred_element_type=jnp.float32))
    ph = jax.nn.relu(jnp.dot(mh, wh[...], preferred_element_type=jnp.float32))
    z = jax.nn.sigmoid(px[:, :D] + ph[:, :D])
    r = jax.nn.sigmoid(px[:, D:2 * D] + ph[:, D:2 * D])
    z_o[...] = z
    rh_o[...] = r * hp[...]
    pxh_o[...] = px[:, 2 * D:]
    inv_o[...] = inv


def _tc1(sx, sh, cnt16, hp, wx, wh):
    blk = lambda i: (i, 0)
    w0 = lambda i: (0, 0)
    return pl.pallas_call(
        _tc1_body,
        grid=(NP // BR,),
        in_specs=[
            pl.BlockSpec((BR, D), blk),
            pl.BlockSpec((BR, D), blk),
            pl.BlockSpec((NS, BR), lambda i: (0, i)),
            pl.BlockSpec((BR, D), blk),
            pl.BlockSpec((D, 3 * D), w0),
            pl.BlockSpec((D, 2 * D), w0),
        ],
        out_specs=[pl.BlockSpec((BR, D), blk)] * 3 + [pl.BlockSpec((BR, 1), blk)],
        out_shape=[jax.ShapeDtypeStruct((N, D), jnp.float32)] * 3
        + [jax.ShapeDtypeStruct((N, 1), jnp.float32)],
    )(sx, sh, cnt16, hp, wx, wh)


def _tc2_body(sa, sb, inv, hp, z, pxh, whh, out):
    m = (sa[...] + sb[...]) * inv[...]
    ph = jax.nn.relu(jnp.dot(m, whh[...], preferred_element_type=jnp.float32))
    ht = jnp.tanh(pxh[...] + ph)
    zz = z[...]
    out[...] = zz * hp[...] + (1.0 - zz) * ht


def _tc2(sa, sb, inv, hp, z, pxh, whh):
    blk = lambda i: (i, 0)
    w0 = lambda i: (0, 0)
    return pl.pallas_call(
        _tc2_body,
        grid=(NP // BR,),
        in_specs=[
            pl.BlockSpec((BR, D), blk),
            pl.BlockSpec((BR, D), blk),
            pl.BlockSpec((BR, 1), blk),
            pl.BlockSpec((BR, D), blk),
            pl.BlockSpec((BR, D), blk),
            pl.BlockSpec((BR, D), blk),
            pl.BlockSpec((D, D), w0),
        ],
        out_specs=pl.BlockSpec((BR, D), blk),
        out_shape=jax.ShapeDtypeStruct((N, D), jnp.float32),
    )(sa, sb, inv, hp, z, pxh, whh)


def kernel(x, edge_index, h_prev, W_xz, W_hz, W_xr, W_hr, W_xh, W_hh):
    loops = jnp.arange(N, dtype=jnp.int32)
    pad = EPAD - E2
    pk = jnp.arange(pad, dtype=jnp.int32)
    row = jnp.concatenate([edge_index[0], loops, pk % N])
    col = jnp.concatenate([edge_index[1], loops, N + pk % (NP - N)])

    sx, sh, cnt16 = _seg1(x, h_prev, row, col)
    wx = jnp.concatenate([W_xz, W_xr, W_xh], axis=1)
    wh = jnp.concatenate([W_hz, W_hr], axis=1)

    z, rh, pxh, inv = _tc1(sx, sh, cnt16, h_prev, wx, wh)

    s2a, s2b = _seg2(rh, row, col)

    return _tc2(s2a, s2b, inv, h_prev, z, pxh, W_hh)


# speedup vs baseline: 1.1747x; 1.0002x over previous
import jax
import jax.numpy as jnp
from jax import lax
from jax.experimental import pallas as pl
from jax.experimental.pallas import tpu as pltpu
from jax.experimental.pallas import tpu_sc as plsc

N = 10000
D = 128
E = 320000

NC = 2
NS = 16
B = 128
E2 = E + N
EPAD = -(-E2 // (NS * B * 4)) * (NS * B * 4)
NP = 10240
RPT = NP // NS
NBLK1 = EPAD // (NS * B)
NBLK2 = EPAD // (NC * NS * B)

_MESH = plsc.VectorSubcoreMesh(core_axis_name="c", subcore_axis_name="s",
                               num_cores=NC, num_subcores=NS)


def _zero_rows(rows):
    z16 = jnp.zeros((16,), jnp.float32)

    def zr(i, _):
        rows[i // (D // 16), pl.ds((i % (D // 16)) * 16, 16)] = z16
        return 0

    lax.fori_loop(0, B * D // 16, zr, 0)


def _init_acc(rows, acc, sid):
    for j in range(RPT // B):
        pltpu.sync_copy(rows, acc.at[pl.ds(sid * RPT + j * B, B)])


def _edge_loop(val_hbm, src_hbm, gbase, dst_hbm, dbase,
               S, SC_, D2, SD, RW, SR, acc, nblk, cnt_ctx=None):
    def req_s(b, p):
        pltpu.async_copy(src_hbm.at[pl.ds(gbase + b * B, B)], S[p], SC_[p])

    def req_d(b, p):
        pltpu.async_copy(dst_hbm.at[pl.ds(dbase + b * B, B)], D2[p], SD[p])

    def gather(b, p, r):
        pltpu.make_async_copy(src_hbm.at[pl.ds(gbase + b * B, B)],
                              S[p], SC_[p]).wait()
        pltpu.async_copy(val_hbm.at[S[p]], RW[r], SR[r])

    o16 = jnp.ones((16,), jnp.float32)

    def scat(b, p, r):
        pltpu.make_async_copy(val_hbm.at[S[p]], RW[r], SR[r]).wait()
        pltpu.make_async_copy(dst_hbm.at[pl.ds(dbase + b * B, B)],
                              D2[r], SD[r]).wait()
        pltpu.sync_copy(RW[r], acc.at[D2[r]], add=True)
        if cnt_ctx is not None:
            for e in range(B // 16):
                plsc.addupdate_scatter(
                    cnt_ctx, [D2[r][pl.ds(e * 16, 16)]], o16)

    for b in range(4):
        req_s(b, b)
    for b in range(2):
        req_d(b, b)
    gather(0, 0, 0)

    main = nblk // 4 - 1

    def quad(j, _):
        for k in range(4):
            b = 4 * j + k
            gather(b + 1, (k + 1) % 4, (k + 1) % 2)
            scat(b, k, k % 2)
            req_s(b + 4, k)
            req_d(b + 2, k % 2)
        return 0

    lax.fori_loop(0, main, quad, 0)

    for bb in range(4 * main, nblk):
        k = bb % 4
        if bb + 1 < nblk:
            gather(bb + 1, (k + 1) % 4, (k + 1) % 2)
        scat(bb, k, k % 2)
        if bb + 4 < nblk:
            req_s(bb + 4, k)
        if bb + 2 < nblk:
            req_d(bb + 2, k % 2)


def _seg1_body(x_hbm, h_hbm, src_hbm, dst_hbm, sx_hbm, sh_hbm, cnt_hbm,
               s0, s1, s2, s3, d0, d1, rows0, rows1,
               cnt_v, acc,
               sc0, sc1, sc2, sc3, sd0, sd1, sr0, sr1):
    cid = lax.axis_index("c")
    sid = lax.axis_index("s")
    z16 = jnp.zeros((16,), jnp.float32)

    _zero_rows(rows0)
    _init_acc(rows0, acc, sid)

    @pl.when(cid == 0)
    def _():
        def zc(i, _):
            cnt_v[pl.ds(i * 16, 16)] = z16
            return 0

        lax.fori_loop(0, NP // 16, zc, 0)

    plsc.subcore_barrier()

    tb = sid * (NBLK1 * B)
    bufs = ([s0, s1, s2, s3], [sc0, sc1, sc2, sc3],
            [d0, d1], [sd0, sd1], [rows0, rows1], [sr0, sr1])

    @pl.when(cid == 0)
    def _():
        _edge_loop(x_hbm, src_hbm, tb, dst_hbm, tb, *bufs,
                   acc, NBLK1, cnt_ctx=cnt_v)
        pltpu.sync_copy(cnt_v, cnt_hbm.at[sid])

    @pl.when(cid == 1)
    def _():
        _edge_loop(h_hbm, src_hbm, tb, dst_hbm, tb, *bufs,
                   acc, NBLK1)

    plsc.subcore_barrier()

    @pl.when(cid == 0)
    def _():
        pltpu.sync_copy(acc.at[pl.ds(sid * RPT, RPT)],
                        sx_hbm.at[pl.ds(sid * RPT, RPT)])

    @pl.when(cid == 1)
    def _():
        pltpu.sync_copy(acc.at[pl.ds(sid * RPT, RPT)],
                        sh_hbm.at[pl.ds(sid * RPT, RPT)])


_seg1 = pl.kernel(
    _seg1_body,
    out_type=[jax.ShapeDtypeStruct((NP, D), jnp.float32),
              jax.ShapeDtypeStruct((NP, D), jnp.float32),
              jax.ShapeDtypeStruct((NS, NP), jnp.float32)],
    mesh=_MESH,
    scratch_types=[
        pltpu.VMEM((B,), jnp.int32),
        pltpu.VMEM((B,), jnp.int32),
        pltpu.VMEM((B,), jnp.int32),
        pltpu.VMEM((B,), jnp.int32),
        pltpu.VMEM((B,), jnp.int32),
        pltpu.VMEM((B,), jnp.int32),
        pltpu.VMEM((B, D), jnp.float32),
        pltpu.VMEM((B, D), jnp.float32),
        pltpu.VMEM((NP,), jnp.float32),
        pltpu.VMEM_SHARED((NP, D), jnp.float32),
    ] + [pltpu.SemaphoreType.DMA] * 8,
    compiler_params=pltpu.CompilerParams(needs_layout_passes=False),
)


def _seg2_body(rh_hbm, src_hbm, dst_hbm, sa_hbm, sb_hbm,
               s0, s1, s2, s3, d0, d1, rows0, rows1, acc,
               sc0, sc1, sc2, sc3, sd0, sd1, sr0, sr1):
    cid = lax.axis_index("c")
    sid = lax.axis_index("s")

    _zero_rows(rows0)
    _init_acc(rows0, acc, sid)

    plsc.subcore_barrier()

    wbase = (cid * NS + sid) * (NBLK2 * B)
    _edge_loop(rh_hbm, src_hbm, wbase, dst_hbm, wbase,
               [s0, s1, s2, s3], [sc0, sc1, sc2, sc3],
               [d0, d1], [sd0, sd1], [rows0, rows1], [sr0, sr1],
               acc, NBLK2)

    plsc.subcore_barrier()

    @pl.when(cid == 0)
    def _():
        pltpu.sync_copy(acc.at[pl.ds(sid * RPT, RPT)],
                        sa_hbm.at[pl.ds(sid * RPT, RPT)])

    @pl.when(cid == 1)
    def _():
        pltpu.sync_copy(acc.at[pl.ds(sid * RPT, RPT)],
                        sb_hbm.at[pl.ds(sid * RPT, RPT)])


_seg2 = pl.kernel(
    _seg2_body,
    out_type=[jax.ShapeDtypeStruct((NP, D), jnp.float32),
              jax.ShapeDtypeStruct((NP, D), jnp.float32)],
    mesh=_MESH,
    scratch_types=[
        pltpu.VMEM((B,), jnp.int32),
        pltpu.VMEM((B,), jnp.int32),
        pltpu.VMEM((B,), jnp.int32),
        pltpu.VMEM((B,), jnp.int32),
        pltpu.VMEM((B,), jnp.int32),
        pltpu.VMEM((B,), jnp.int32),
        pltpu.VMEM((B, D), jnp.float32),
        pltpu.VMEM((B, D), jnp.float32),
        pltpu.VMEM_SHARED((NP, D), jnp.float32),
    ] + [pltpu.SemaphoreType.DMA] * 8,
    compiler_params=pltpu.CompilerParams(needs_layout_passes=False),
)

BR = 1024


def _tc1a_body(sx, sh, cnt, hp, wxr, whr, rh_o, inv_o):
    cs = jnp.sum(jnp.transpose(cnt[...]), axis=1, keepdims=True)
    inv = 1.0 / jnp.maximum(cs, 1.0)
    mx = sx[...] * inv
    mh = sh[...] * inv
    px = jax.nn.relu(jnp.dot(mx, wxr[...], preferred_element_type=jnp.float32))
    ph = jax.nn.relu(jnp.dot(mh, whr[...], preferred_element_type=jnp.float32))
    r = jax.nn.sigmoid(px + ph)
    rh_o[...] = r * hp[...]
    inv_o[...] = inv


def _tc1a(sx, sh, cnt16, hp, wxr, whr):
    blk = lambda i: (i, 0)
    w0 = lambda i: (0, 0)
    return pl.pallas_call(
        _tc1a_body,
        grid=(NP // BR,),
        in_specs=[
            pl.BlockSpec((BR, D), blk),
            pl.BlockSpec((BR, D), blk),
            pl.BlockSpec((NS, BR), lambda i: (0, i)),
            pl.BlockSpec((BR, D), blk),
            pl.BlockSpec((D, D), w0),
            pl.BlockSpec((D, D), w0),
        ],
        out_specs=[pl.BlockSpec((BR, D), blk), pl.BlockSpec((BR, 1), blk)],
        out_shape=[jax.ShapeDtypeStruct((N, D), jnp.float32),
                   jax.ShapeDtypeStruct((N, 1), jnp.float32)],
    )(sx, sh, cnt16, hp, wxr, whr)


def _tc1b_body(sx, sh, inv, wxzh, whz, z_o, pxh_o):
    mx = sx[...] * inv[...]
    mh = sh[...] * inv[...]
    px = jax.nn.relu(jnp.dot(mx, wxzh[...],
                             preferred_element_type=jnp.float32))
    ph = jax.nn.relu(jnp.dot(mh, whz[...], preferred_element_type=jnp.float32))
    z_o[...] = jax.nn.sigmoid(px[:, :D] + ph)
    pxh_o[...] = px[:, D:]


def _tc1b(sx, sh, inv, wxzh, whz):
    blk = lambda i: (i, 0)
    w0 = lambda i: (0, 0)
    return pl.pallas_call(
        _tc1b_body,
        grid=(NP // BR,),
        in_specs=[
            pl.BlockSpec((BR, D), blk),
            pl.BlockSpec((BR, D), blk),
            pl.BlockSpec((BR, 1), blk),
            pl.BlockSpec((D, 2 * D), w0),
            pl.BlockSpec((D, D), w0),
        ],
        out_specs=[pl.BlockSpec((BR, D), blk)] * 2,
        out_shape=[jax.ShapeDtypeStruct((N, D), jnp.float32)] * 2,
    )(sx, sh, inv, wxzh, whz)


def _tc2_body(sa, sb, inv, hp, z, pxh, whh, out):
    m = (sa[...] + sb[...]) * inv[...]
    ph = jax.nn.relu(jnp.dot(m, whh[...], preferred_element_type=jnp.float32))
    ht = jnp.tanh(pxh[...] + ph)
    zz = z[...]
    out[...] = zz * hp[...] + (1.0 - zz) * ht


def _tc2(sa, sb, inv, hp, z, pxh, whh):
    blk = lambda i: (i, 0)
    w0 = lambda i: (0, 0)
    return pl.pallas_call(
        _tc2_body,
        grid=(NP // BR,),
        in_specs=[
            pl.BlockSpec((BR, D), blk),
            pl.BlockSpec((BR, D), blk),
            pl.BlockSpec((BR, 1), blk),
            pl.BlockSpec((BR, D), blk),
            pl.BlockSpec((BR, D), blk),
            pl.BlockSpec((BR, D), blk),
            pl.BlockSpec((D, D), w0),
        ],
        out_specs=pl.BlockSpec((BR, D), blk),
        out_shape=jax.ShapeDtypeStruct((N, D), jnp.float32),
    )(sa, sb, inv, hp, z, pxh, whh)


def kernel(x, edge_index, h_prev, W_xz, W_hz, W_xr, W_hr, W_xh, W_hh):
    loops = jnp.arange(N, dtype=jnp.int32)
    pad = EPAD - E2
    pk = jnp.arange(pad, dtype=jnp.int32)
    row = jnp.concatenate([edge_index[0], loops, pk % N])
    col = jnp.concatenate([edge_index[1], loops, N + pk % (NP - N)])

    sx, sh, cnt16 = _seg1(x, h_prev, row, col)
    wxzh = jnp.concatenate([W_xz, W_xh], axis=1)

    rh, inv = _tc1a(sx, sh, cnt16, h_prev, W_xr, W_hr)

    s2a, s2b = _seg2(rh, row, col)
    z, pxh = _tc1b(sx, sh, inv, wxzh, W_hz)

    return _tc2(s2a, s2b, inv, h_prev, z, pxh, W_hh)
